# Initial kernel scaffold; baseline (speedup 1.0000x reference)
import functools

import jax
import jax.numpy as jnp
from jax import lax
from jax.experimental import pallas as pl
from jax.experimental.pallas import tpu as pltpu
from jax.experimental.pallas import tpu_sc as plsc

N = 10000
E = 320000
H = 128

NC = 2
NS = 16
NW = NC * NS
EPW = E // NW
CH = 80
NCH = EPW // CH
RPT = N // NS

_MESH = plsc.VectorSubcoreMesh(core_axis_name="c", subcore_axis_name="s")


def _sc_agg_body(h_hbm, src_hbm, dst_hbm, ones_hbm, zrow_hbm, zdeg_hbm,
                 agg_out, deg_out, src_v, dst_v, rows_v, ones_v,
                 agg_sh, deg_sh, sem, *, with_deg):
    c = lax.axis_index("c")
    s = lax.axis_index("s")
    wid = s * NC + c
    pltpu.sync_copy(zrow_hbm, agg_sh.at[pl.ds(s * RPT, RPT)])
    if with_deg:
        pltpu.sync_copy(zdeg_hbm, deg_sh.at[pl.ds(s * RPT, RPT)])
        pltpu.sync_copy(ones_hbm, ones_v)
    pltpu.sync_copy(src_hbm.at[wid], src_v)
    pltpu.sync_copy(dst_hbm.at[wid], dst_v)
    plsc.subcore_barrier()

    def body(j, carry):
        pltpu.async_copy(h_hbm.at[src_v.at[j]], rows_v, sem).wait()
        pltpu.sync_copy(rows_v, agg_sh.at[dst_v.at[j]], add=True)
        if with_deg:
            pltpu.sync_copy(ones_v, deg_sh.at[dst_v.at[j]], add=True)
        return carry

    lax.fori_loop(0, NCH, body, 0)
    plsc.subcore_barrier()
    pltpu.sync_copy(agg_sh.at[pl.ds(s * RPT, RPT)],
                    agg_out.at[c, pl.ds(s * RPT, RPT)])
    if with_deg:
        pltpu.sync_copy(deg_sh.at[pl.ds(s * RPT, RPT)],
                        deg_out.at[c, pl.ds(s * RPT, RPT)])


def _make_sc_kernel(with_deg):
    out_type = [jax.ShapeDtypeStruct((NC, N, H), jnp.float32)]
    if with_deg:
        out_type.append(jax.ShapeDtypeStruct((NC, N, 16), jnp.float32))
    scratch = [
        pltpu.VMEM((NCH, CH), jnp.int32),
        pltpu.VMEM((NCH, CH), jnp.int32),
        pltpu.VMEM((CH, H), jnp.float32),
        pltpu.VMEM((CH, 16), jnp.float32),
        pltpu.VMEM_SHARED((N, H), jnp.float32),
        pltpu.VMEM_SHARED((N, 16), jnp.float32),
        pltpu.SemaphoreType.DMA,
    ]

    def body(h_hbm, src_hbm, dst_hbm, ones_hbm, zrow_hbm, zdeg_hbm, *rest):
        if with_deg:
            agg_out, deg_out = rest[0], rest[1]
            scr = rest[2:]
        else:
            agg_out, deg_out = rest[0], None
            scr = rest[1:]
        _sc_agg_body(h_hbm, src_hbm, dst_hbm, ones_hbm, zrow_hbm, zdeg_hbm,
                     agg_out, deg_out, *scr, with_deg=with_deg)

    return pl.kernel(body, out_type=out_type if with_deg else out_type[0],
                     mesh=_MESH, scratch_types=scratch)


_sc_agg_deg = _make_sc_kernel(True)
_sc_agg = _make_sc_kernel(False)


def _tc_combine_body(h_ref, agg_ref, deg_ref, ws_ref, wn_ref, b_ref, out_ref,
                     *, relu):
    h = h_ref[...]
    agg = agg_ref[0] + agg_ref[1]
    deg = deg_ref[0] + deg_ref[1]
    recip = 1.0 / jnp.maximum(deg, 1.0)
    hn = agg * recip[:, 0:1]
    out = (jnp.dot(h, ws_ref[...], preferred_element_type=jnp.float32)
           + jnp.dot(hn, wn_ref[...], preferred_element_type=jnp.float32)
           + b_ref[...])
    out_ref[...] = jnp.maximum(out, 0.0) if relu else out


_BN = 1000


def _tc_combine(h, agg, deg, ws, wn, b, relu):
    grid = (N // _BN,)
    return pl.pallas_call(
        functools.partial(_tc_combine_body, relu=relu),
        grid=grid,
        in_specs=[
            pl.BlockSpec((_BN, H), lambda i: (i, 0)),
            pl.BlockSpec((NC, _BN, H), lambda i: (0, i, 0)),
            pl.BlockSpec((NC, _BN, 16), lambda i: (0, i, 0)),
            pl.BlockSpec((H, H), lambda i: (0, 0)),
            pl.BlockSpec((H, H), lambda i: (0, 0)),
            pl.BlockSpec((1, H), lambda i: (0, 0)),
        ],
        out_specs=pl.BlockSpec((_BN, H), lambda i: (i, 0)),
        out_shape=jax.ShapeDtypeStruct((N, H), jnp.float32),
    )(h, agg, deg, ws, wn, b.reshape(1, H))


def kernel(in_feat, edge_index, W0s, W0n, b0, W1s, W1n, b1, W2s, W2n, b2):
    src = edge_index[0].astype(jnp.int32).reshape(NW, NCH, CH)
    dst = edge_index[1].astype(jnp.int32).reshape(NW, NCH, CH)
    ones = jnp.ones((CH, 16), jnp.float32)
    zrow = jnp.zeros((RPT, H), jnp.float32)
    zdeg = jnp.zeros((RPT, 16), jnp.float32)

    agg0, deg = _sc_agg_deg(in_feat, src, dst, ones, zrow, zdeg)
    h1 = _tc_combine(in_feat, agg0, deg, W0s, W0n, b0, relu=True)
    agg1 = _sc_agg(h1, src, dst, ones, zrow, zdeg)
    h2 = _tc_combine(h1, agg1, deg, W1s, W1n, b1, relu=True)
    agg2 = _sc_agg(h2, src, dst, ones, zrow, zdeg)
    return _tc_combine(h2, agg2, deg, W2s, W2n, b2, relu=False)


# speedup vs baseline: 7.5982x; 7.5982x over previous
import functools

import jax
import jax.numpy as jnp
from jax import lax
from jax.experimental import pallas as pl
from jax.experimental.pallas import tpu as pltpu
from jax.experimental.pallas import tpu_sc as plsc

N = 10000
E = 320000
H = 128

NC = 2
NS = 16
NW = NC * NS
EPW = E // NW
CH = 125
GC = 8
NG = EPW // (GC * CH)
RPT = 640
N_PAD = NS * RPT
RB = 128
NRB = RPT // RB

_MESH = plsc.VectorSubcoreMesh(core_axis_name="c", subcore_axis_name="s")


def _sc_agg_body(h_hbm, src_hbm, dst_hbm, zrow_hbm, agg_out,
                 src_v, dst_v, rows_v, zbuf, agg_sh, sem):
    c = lax.axis_index("c")
    s = lax.axis_index("s")
    wid = s * NC + c
    pltpu.sync_copy(zrow_hbm, zbuf)
    for k in range(NRB):
        pltpu.sync_copy(zbuf, agg_sh.at[pl.ds(s * RPT + k * RB, RB)])
    plsc.subcore_barrier()

    def body(g, carry):
        pltpu.sync_copy(src_hbm.at[wid, g], src_v)
        pltpu.sync_copy(dst_hbm.at[wid, g], dst_v)
        for j in range(GC):
            pltpu.async_copy(h_hbm.at[src_v.at[j]], rows_v, sem).wait()
            pltpu.sync_copy(rows_v, agg_sh.at[dst_v.at[j]], add=True)
        return carry

    lax.fori_loop(0, NG, body, 0)
    plsc.subcore_barrier()
    for k in range(NRB):
        pltpu.sync_copy(agg_sh.at[pl.ds(s * RPT + k * RB, RB)], zbuf)
        pltpu.sync_copy(zbuf, agg_out.at[c, pl.ds(s * RPT + k * RB, RB)])


_sc_agg = pl.kernel(
    _sc_agg_body,
    out_type=jax.ShapeDtypeStruct((NC, N_PAD, H), jnp.float32),
    mesh=_MESH,
    scratch_types=[
        pltpu.VMEM((GC, CH), jnp.int32),
        pltpu.VMEM((GC, CH), jnp.int32),
        pltpu.VMEM((CH, H), jnp.float32),
        pltpu.VMEM((RB, H), jnp.float32),
        pltpu.VMEM_SHARED((N_PAD, H), jnp.float32),
        pltpu.SemaphoreType.DMA,
    ],
)


CHD = 128
GCD = 8
NGD = 10
E_PAD = NW * NGD * GCD * CHD


def _sc_deg_body(dst_hbm, ones_hbm, zrow_hbm, deg_out,
                 dst_v, rows_v, zbuf, deg_sh):
    c = lax.axis_index("c")
    s = lax.axis_index("s")
    wid = s * NC + c
    pltpu.sync_copy(zrow_hbm, zbuf)
    for k in range(NRB):
        pltpu.sync_copy(zbuf, deg_sh.at[pl.ds(s * RPT + k * RB, RB)])
    pltpu.sync_copy(ones_hbm, rows_v)
    plsc.subcore_barrier()

    def body(g, carry):
        pltpu.sync_copy(dst_hbm.at[wid, g], dst_v)
        for j in range(GCD):
            pltpu.sync_copy(rows_v, deg_sh.at[dst_v.at[j]], add=True)
        return carry

    lax.fori_loop(0, NGD, body, 0)
    plsc.subcore_barrier()
    for k in range(NRB):
        pltpu.sync_copy(deg_sh.at[pl.ds(s * RPT + k * RB, RB)], zbuf)
        pltpu.sync_copy(zbuf, deg_out.at[c, pl.ds(s * RPT + k * RB, RB)])


_sc_deg = pl.kernel(
    _sc_deg_body,
    out_type=jax.ShapeDtypeStruct((NC, N_PAD, H), jnp.float32),
    mesh=_MESH,
    scratch_types=[
        pltpu.VMEM((GCD, CHD), jnp.int32),
        pltpu.VMEM((CHD, H), jnp.float32),
        pltpu.VMEM((RB, H), jnp.float32),
        pltpu.VMEM_SHARED((N_PAD, H), jnp.float32),
    ],
)


def _tc_combine_body(h_ref, agg_ref, deg_ref, ws_ref, wn_ref, b_ref, out_ref,
                     *, relu):
    h = h_ref[...]
    agg = agg_ref[0] + agg_ref[1]
    deg = deg_ref[0, :, 0:1] + deg_ref[1, :, 0:1]
    recip = 1.0 / jnp.maximum(deg, 1.0)
    hn = agg * recip
    out = (jnp.dot(h, ws_ref[...], preferred_element_type=jnp.float32)
           + jnp.dot(hn, wn_ref[...], preferred_element_type=jnp.float32)
           + b_ref[...])
    out_ref[...] = jnp.maximum(out, 0.0) if relu else out


_BN = 1000


def _tc_combine(h, agg, deg, ws, wn, b, relu):
    grid = (N // _BN,)
    return pl.pallas_call(
        functools.partial(_tc_combine_body, relu=relu),
        grid=grid,
        in_specs=[
            pl.BlockSpec((_BN, H), lambda i: (i, 0)),
            pl.BlockSpec((NC, _BN, H), lambda i: (0, i, 0)),
            pl.BlockSpec((NC, _BN, H), lambda i: (0, i, 0)),
            pl.BlockSpec((H, H), lambda i: (0, 0)),
            pl.BlockSpec((H, H), lambda i: (0, 0)),
            pl.BlockSpec((1, H), lambda i: (0, 0)),
        ],
        out_specs=pl.BlockSpec((_BN, H), lambda i: (i, 0)),
        out_shape=jax.ShapeDtypeStruct((N, H), jnp.float32),
    )(h, agg, deg, ws, wn, b.reshape(1, H))


def kernel(in_feat, edge_index, W0s, W0n, b0, W1s, W1n, b1, W2s, W2n, b2):
    src_flat = edge_index[0].astype(jnp.int32)
    dst_flat = edge_index[1].astype(jnp.int32)
    src = src_flat.reshape(NW, NG, GC, CH)
    dst = dst_flat.reshape(NW, NG, GC, CH)
    dst_pad = jnp.concatenate(
        [dst_flat, jnp.full((E_PAD - E,), N_PAD - 1, jnp.int32)]
    ).reshape(NW, NGD, GCD, CHD)
    zrow = jnp.zeros((RB, H), jnp.float32)
    ones = jnp.ones((CHD, H), jnp.float32)

    deg = _sc_deg(dst_pad, ones, zrow)
    agg0 = _sc_agg(in_feat, src, dst, zrow)
    h1 = _tc_combine(in_feat, agg0, deg, W0s, W0n, b0, relu=True)
    agg1 = _sc_agg(h1, src, dst, zrow)
    h2 = _tc_combine(h1, agg1, deg, W1s, W1n, b1, relu=True)
    agg2 = _sc_agg(h2, src, dst, zrow)
    return _tc_combine(h2, agg2, deg, W2s, W2n, b2, relu=False)


# speedup vs baseline: 9.8880x; 1.3014x over previous
import functools

import jax
import jax.numpy as jnp
from jax import lax
from jax.experimental import pallas as pl
from jax.experimental.pallas import tpu as pltpu
from jax.experimental.pallas import tpu_sc as plsc

N = 10000
E = 320000
H = 128

NC = 2
NS = 16
NW = NC * NS
EPW = E // NW
CH = 125
GC = 8
NG = EPW // (GC * CH)
RPT = 640
N_PAD = NS * RPT
RB = 64
NRB = RPT // RB

_MESH = plsc.VectorSubcoreMesh(core_axis_name="c", subcore_axis_name="s")


def _sc_agg_body(h_hbm, src_hbm, dst_hbm, zrow_hbm, agg_out,
                 src_v, dst_v, rows_a, rows_b, zbuf, agg_sh,
                 sg_a, sg_b, ss_a, ss_b):
    c = lax.axis_index("c")
    s = lax.axis_index("s")
    wid = s * NC + c
    pltpu.sync_copy(zrow_hbm, zbuf)
    for k in range(NRB):
        pltpu.sync_copy(zbuf, agg_sh.at[pl.ds(s * RPT + k * RB, RB)])
    plsc.subcore_barrier()

    bufs = (rows_a, rows_b)
    sgs = (sg_a, sg_b)
    sss = (ss_a, ss_b)

    def body(g, carry):
        pltpu.sync_copy(src_hbm.at[wid, g], src_v)
        pltpu.sync_copy(dst_hbm.at[wid, g], dst_v)
        gat = [None, None]
        scat = [None, None]
        gat[0] = pltpu.async_copy(h_hbm.at[src_v.at[0]], bufs[0], sgs[0])
        for j in range(GC):
            p = j % 2
            q = (j + 1) % 2
            if j + 1 < GC:
                if scat[q] is not None:
                    scat[q].wait()
                gat[q] = pltpu.async_copy(h_hbm.at[src_v.at[j + 1]],
                                          bufs[q], sgs[q])
            gat[p].wait()
            scat[p] = pltpu.async_copy(bufs[p], agg_sh.at[dst_v.at[j]],
                                       sss[p], add=True)
        scat[0].wait()
        scat[1].wait()
        return carry

    lax.fori_loop(0, NG, body, 0)
    plsc.subcore_barrier()
    for k in range(NRB):
        pltpu.sync_copy(agg_sh.at[pl.ds(s * RPT + k * RB, RB)], zbuf)
        pltpu.sync_copy(zbuf, agg_out.at[c, pl.ds(s * RPT + k * RB, RB)])


_sc_agg = pl.kernel(
    _sc_agg_body,
    out_type=jax.ShapeDtypeStruct((NC, N_PAD, H), jnp.float32),
    mesh=_MESH,
    scratch_types=[
        pltpu.VMEM((GC, CH), jnp.int32),
        pltpu.VMEM((GC, CH), jnp.int32),
        pltpu.VMEM((CH, H), jnp.float32),
        pltpu.VMEM((CH, H), jnp.float32),
        pltpu.VMEM((RB, H), jnp.float32),
        pltpu.VMEM_SHARED((N_PAD, H), jnp.float32),
        pltpu.SemaphoreType.DMA,
        pltpu.SemaphoreType.DMA,
        pltpu.SemaphoreType.DMA,
        pltpu.SemaphoreType.DMA,
    ],
)


CHD = 128
GCD = 8
NGD = 10
E_PAD = NW * NGD * GCD * CHD


def _sc_deg_body(dst_hbm, ones_hbm, zrow_hbm, deg_out,
                 dst_v, rows_v, zbuf, deg_sh):
    c = lax.axis_index("c")
    s = lax.axis_index("s")
    wid = s * NC + c
    pltpu.sync_copy(zrow_hbm, zbuf)
    for k in range(NRB):
        pltpu.sync_copy(zbuf, deg_sh.at[pl.ds(s * RPT + k * RB, RB)])
    pltpu.sync_copy(ones_hbm, rows_v)
    plsc.subcore_barrier()

    def body(g, carry):
        pltpu.sync_copy(dst_hbm.at[wid, g], dst_v)
        for j in range(GCD):
            pltpu.sync_copy(rows_v, deg_sh.at[dst_v.at[j]], add=True)
        return carry

    lax.fori_loop(0, NGD, body, 0)
    plsc.subcore_barrier()
    for k in range(NRB):
        pltpu.sync_copy(deg_sh.at[pl.ds(s * RPT + k * RB, RB)], zbuf)
        pltpu.sync_copy(zbuf, deg_out.at[c, pl.ds(s * RPT + k * RB, RB)])


_sc_deg = pl.kernel(
    _sc_deg_body,
    out_type=jax.ShapeDtypeStruct((NC, N_PAD, H), jnp.float32),
    mesh=_MESH,
    scratch_types=[
        pltpu.VMEM((GCD, CHD), jnp.int32),
        pltpu.VMEM((CHD, H), jnp.float32),
        pltpu.VMEM((RB, H), jnp.float32),
        pltpu.VMEM_SHARED((N_PAD, H), jnp.float32),
    ],
)


def _tc_combine_body(h_ref, agg_ref, deg_ref, ws_ref, wn_ref, b_ref, out_ref,
                     *, relu):
    h = h_ref[...]
    agg = agg_ref[0] + agg_ref[1]
    deg = deg_ref[0, :, 0:1] + deg_ref[1, :, 0:1]
    recip = 1.0 / jnp.maximum(deg, 1.0)
    hn = agg * recip
    out = (jnp.dot(h, ws_ref[...], preferred_element_type=jnp.float32)
           + jnp.dot(hn, wn_ref[...], preferred_element_type=jnp.float32)
           + b_ref[...])
    out_ref[...] = jnp.maximum(out, 0.0) if relu else out


_BN = 1000


def _tc_combine(h, agg, deg, ws, wn, b, relu):
    grid = (N // _BN,)
    return pl.pallas_call(
        functools.partial(_tc_combine_body, relu=relu),
        grid=grid,
        in_specs=[
            pl.BlockSpec((_BN, H), lambda i: (i, 0)),
            pl.BlockSpec((NC, _BN, H), lambda i: (0, i, 0)),
            pl.BlockSpec((NC, _BN, H), lambda i: (0, i, 0)),
            pl.BlockSpec((H, H), lambda i: (0, 0)),
            pl.BlockSpec((H, H), lambda i: (0, 0)),
            pl.BlockSpec((1, H), lambda i: (0, 0)),
        ],
        out_specs=pl.BlockSpec((_BN, H), lambda i: (i, 0)),
        out_shape=jax.ShapeDtypeStruct((N, H), jnp.float32),
    )(h, agg, deg, ws, wn, b.reshape(1, H))


def kernel(in_feat, edge_index, W0s, W0n, b0, W1s, W1n, b1, W2s, W2n, b2):
    src_flat = edge_index[0].astype(jnp.int32)
    dst_flat = edge_index[1].astype(jnp.int32)
    src = src_flat.reshape(NW, NG, GC, CH)
    dst = dst_flat.reshape(NW, NG, GC, CH)
    dst_pad = jnp.concatenate(
        [dst_flat, jnp.full((E_PAD - E,), N_PAD - 1, jnp.int32)]
    ).reshape(NW, NGD, GCD, CHD)
    zrow = jnp.zeros((RB, H), jnp.float32)
    ones = jnp.ones((CHD, H), jnp.float32)

    deg = _sc_deg(dst_pad, ones, zrow)
    agg0 = _sc_agg(in_feat, src, dst, zrow)
    h1 = _tc_combine(in_feat, agg0, deg, W0s, W0n, b0, relu=True)
    agg1 = _sc_agg(h1, src, dst, zrow)
    h2 = _tc_combine(h1, agg1, deg, W1s, W1n, b1, relu=True)
    agg2 = _sc_agg(h2, src, dst, zrow)
    return _tc_combine(h2, agg2, deg, W2s, W2n, b2, relu=False)


# speedup vs baseline: 10.7559x; 1.0878x over previous
import functools

import jax
import jax.numpy as jnp
from jax import lax
from jax.experimental import pallas as pl
from jax.experimental.pallas import tpu as pltpu
from jax.experimental.pallas import tpu_sc as plsc

N = 10000
E = 320000
H = 128

NC = 2
NS = 16
NW = NC * NS
EPW = E // NW
CH = 125
GC = 20
NG = EPW // (GC * CH)
RPT = 640
N_PAD = NS * RPT
RB = 64
NRB = RPT // RB

_MESH = plsc.VectorSubcoreMesh(core_axis_name="c", subcore_axis_name="s")


def _sc_agg_body(h_hbm, src_hbm, dst_hbm, zrow_hbm, agg_out,
                 src_v, dst_v, rows_a, rows_b, zbuf, agg_sh,
                 sg_a, sg_b, ss_a, ss_b):
    c = lax.axis_index("c")
    s = lax.axis_index("s")
    wid = s * NC + c
    pltpu.sync_copy(zrow_hbm, zbuf)
    for k in range(NRB):
        pltpu.sync_copy(zbuf, agg_sh.at[pl.ds(s * RPT + k * RB, RB)])
    plsc.subcore_barrier()

    bufs = (rows_a, rows_b)
    sgs = (sg_a, sg_b)
    sss = (ss_a, ss_b)

    def body(g, carry):
        pltpu.sync_copy(src_hbm.at[wid, g], src_v)
        pltpu.sync_copy(dst_hbm.at[wid, g], dst_v)
        gat = [None, None]
        scat = [None, None]
        gat[0] = pltpu.async_copy(h_hbm.at[src_v.at[0]], bufs[0], sgs[0])
        for j in range(GC):
            p = j % 2
            q = (j + 1) % 2
            if j + 1 < GC:
                if scat[q] is not None:
                    scat[q].wait()
                gat[q] = pltpu.async_copy(h_hbm.at[src_v.at[j + 1]],
                                          bufs[q], sgs[q])
            gat[p].wait()
            scat[p] = pltpu.async_copy(bufs[p], agg_sh.at[dst_v.at[j]],
                                       sss[p], add=True)
        scat[0].wait()
        scat[1].wait()
        return carry

    lax.fori_loop(0, NG, body, 0)
    plsc.subcore_barrier()
    for k in range(NRB):
        pltpu.sync_copy(agg_sh.at[pl.ds(s * RPT + k * RB, RB)], zbuf)
        pltpu.sync_copy(zbuf, agg_out.at[c, pl.ds(s * RPT + k * RB, RB)])


_sc_agg = pl.kernel(
    _sc_agg_body,
    out_type=jax.ShapeDtypeStruct((NC, N_PAD, H), jnp.float32),
    mesh=_MESH,
    scratch_types=[
        pltpu.VMEM((GC, CH), jnp.int32),
        pltpu.VMEM((GC, CH), jnp.int32),
        pltpu.VMEM((CH, H), jnp.float32),
        pltpu.VMEM((CH, H), jnp.float32),
        pltpu.VMEM((RB, H), jnp.float32),
        pltpu.VMEM_SHARED((N_PAD, H), jnp.float32),
        pltpu.SemaphoreType.DMA,
        pltpu.SemaphoreType.DMA,
        pltpu.SemaphoreType.DMA,
        pltpu.SemaphoreType.DMA,
    ],
)


CHD = 128
GCD = 20
NGD = 4
E_PAD = NW * NGD * GCD * CHD


def _sc_deg_body(dst_hbm, ones_hbm, zrow_hbm, deg_out,
                 dst_v, rows_v, zbuf, deg_sh, sem):
    c = lax.axis_index("c")
    s = lax.axis_index("s")
    wid = s * NC + c
    pltpu.sync_copy(zrow_hbm, zbuf)
    for k in range(NRB):
        pltpu.sync_copy(zbuf, deg_sh.at[pl.ds(s * RPT + k * RB, RB)])
    pltpu.sync_copy(ones_hbm, rows_v)
    plsc.subcore_barrier()

    def body(g, carry):
        pltpu.sync_copy(dst_hbm.at[wid, g], dst_v)
        descs = []
        for j in range(GCD):
            descs.append(pltpu.async_copy(
                rows_v, deg_sh.at[dst_v.at[j]], sem, add=True))
        for d in descs:
            d.wait()
        return carry

    lax.fori_loop(0, NGD, body, 0)
    plsc.subcore_barrier()
    for k in range(NRB):
        pltpu.sync_copy(deg_sh.at[pl.ds(s * RPT + k * RB, RB)], zbuf)
        pltpu.sync_copy(zbuf, deg_out.at[c, pl.ds(s * RPT + k * RB, RB)])


_sc_deg = pl.kernel(
    _sc_deg_body,
    out_type=jax.ShapeDtypeStruct((NC, N_PAD, H), jnp.float32),
    mesh=_MESH,
    scratch_types=[
        pltpu.VMEM((GCD, CHD), jnp.int32),
        pltpu.VMEM((CHD, H), jnp.float32),
        pltpu.VMEM((RB, H), jnp.float32),
        pltpu.VMEM_SHARED((N_PAD, H), jnp.float32),
        pltpu.SemaphoreType.DMA,
    ],
)


def _tc_combine_body(h_ref, agg_ref, deg_ref, ws_ref, wn_ref, b_ref, out_ref,
                     *, relu):
    h = h_ref[...]
    agg = agg_ref[0] + agg_ref[1]
    deg = deg_ref[0, :, 0:1] + deg_ref[1, :, 0:1]
    recip = 1.0 / jnp.maximum(deg, 1.0)
    hn = agg * recip
    out = (jnp.dot(h, ws_ref[...], preferred_element_type=jnp.float32)
           + jnp.dot(hn, wn_ref[...], preferred_element_type=jnp.float32)
           + b_ref[...])
    out_ref[...] = jnp.maximum(out, 0.0) if relu else out


_BN = 1000


def _tc_combine(h, agg, deg, ws, wn, b, relu):
    grid = (N // _BN,)
    return pl.pallas_call(
        functools.partial(_tc_combine_body, relu=relu),
        grid=grid,
        in_specs=[
            pl.BlockSpec((_BN, H), lambda i: (i, 0)),
            pl.BlockSpec((NC, _BN, H), lambda i: (0, i, 0)),
            pl.BlockSpec((NC, _BN, H), lambda i: (0, i, 0)),
            pl.BlockSpec((H, H), lambda i: (0, 0)),
            pl.BlockSpec((H, H), lambda i: (0, 0)),
            pl.BlockSpec((1, H), lambda i: (0, 0)),
        ],
        out_specs=pl.BlockSpec((_BN, H), lambda i: (i, 0)),
        out_shape=jax.ShapeDtypeStruct((N, H), jnp.float32),
    )(h, agg, deg, ws, wn, b.reshape(1, H))


def kernel(in_feat, edge_index, W0s, W0n, b0, W1s, W1n, b1, W2s, W2n, b2):
    src_flat = edge_index[0].astype(jnp.int32)
    dst_flat = edge_index[1].astype(jnp.int32)
    src = src_flat.reshape(NW, NG, GC, CH)
    dst = dst_flat.reshape(NW, NG, GC, CH)
    dst_pad = jnp.concatenate(
        [dst_flat, jnp.full((E_PAD - E,), N_PAD - 1, jnp.int32)]
    ).reshape(NW, NGD, GCD, CHD)
    zrow = jnp.zeros((RB, H), jnp.float32)
    ones = jnp.ones((CHD, H), jnp.float32)

    deg = _sc_deg(dst_pad, ones, zrow)
    agg0 = _sc_agg(in_feat, src, dst, zrow)
    h1 = _tc_combine(in_feat, agg0, deg, W0s, W0n, b0, relu=True)
    agg1 = _sc_agg(h1, src, dst, zrow)
    h2 = _tc_combine(h1, agg1, deg, W1s, W1n, b1, relu=True)
    agg2 = _sc_agg(h2, src, dst, zrow)
    return _tc_combine(h2, agg2, deg, W2s, W2n, b2, relu=False)
